# baseline (speedup 1.0000x reference)
import functools

import jax
import jax.numpy as jnp
import numpy as np
from jax.experimental import pallas as pl
from jax.experimental.pallas import tpu as pltpu

_LAYER_CFG = [
    (1, 32, 8, 1, 256),
    (32, 64, 8, 2, 256),
    (64, 96, 8, 4, 256),
    (96, 128, 12, 4, 120),
    (128, 160, 12, 6, 120),
]

_SEL_CONST = {
    (li, n): np.asarray(jax.random.permutation(
        jax.random.fold_in(jax.random.key(1), li), n))
    for li, n in ((0, 1024), (3, 256))
}


def _elu(x):
    return jnp.where(x > 0, x, jnp.exp(x) - 1.0)


def _pad_to(x, axis, size, value):
    pad = size - x.shape[axis]
    if pad <= 0:
        return x
    widths = [(0, 0)] * x.ndim
    widths[axis] = (0, pad)
    return jnp.pad(x, widths, constant_values=value)


def _knn_body(rep_ref, pts_ref, fts_ref, dw_ref, db_ref, idx_ref, fd_ref,
              *, K, D):
    Pp = rep_ref.shape[1]
    Npp = pts_ref.shape[2]
    rx = rep_ref[0, :, 0:1]
    ry = rep_ref[0, :, 1:2]
    rz = rep_ref[0, :, 2:3]
    px = pts_ref[0, 0:1, :]
    py = pts_ref[0, 1:2, :]
    pz = pts_ref[0, 2:3, :]
    dx = rx - px
    dy = ry - py
    dz = rz - pz
    d2 = dx * dx + dy * dy + dz * dz
    iota_n = jax.lax.broadcasted_iota(jnp.int32, (Pp, Npp), 1)
    lane_k = jax.lax.broadcasted_iota(jnp.int32, (Pp, idx_ref.shape[2]), 1)
    acc = jnp.zeros((Pp, idx_ref.shape[2]), jnp.int32)
    inf = jnp.float32(jnp.inf)
    for k in range(D * K + 1):
        m = jnp.min(d2, axis=1, keepdims=True)
        cand = jnp.where(d2 == m, iota_n, jnp.int32(Npp))
        am = jnp.min(cand, axis=1, keepdims=True)
        if k >= 1 and (k - 1) % D == 0:
            acc = jnp.where(lane_k == (k - 1) // D, am, acc)
        if k < D * K:
            d2 = jnp.where(iota_n == am, inf, d2)
    idx_ref[0] = acc
    fd_ref[0] = _elu(jnp.dot(fts_ref[0], dw_ref[...],
                             preferred_element_type=jnp.float32) + db_ref[...])


def _knn_dense(rep, pts, fts, dense_W, dense_b, K, D):
    B, P, _ = rep.shape
    Np = pts.shape[1]
    C_in = fts.shape[2]
    C_prev = dense_W.shape[1]
    Pp = max(128, ((P + 7) // 8) * 8)
    Npp = max(128, ((Np + 127) // 128) * 128)
    Kp = 16
    repp = _pad_to(rep, 1, Pp, 0.0)
    ptsp = _pad_to(jnp.swapaxes(pts, 1, 2), 2, Npp, 1e30)
    idx, fd = pl.pallas_call(
        functools.partial(_knn_body, K=K, D=D),
        grid=(B,),
        in_specs=[
            pl.BlockSpec((1, Pp, 3), lambda b: (b, 0, 0)),
            pl.BlockSpec((1, 3, Npp), lambda b: (b, 0, 0)),
            pl.BlockSpec((1, Np, C_in), lambda b: (b, 0, 0)),
            pl.BlockSpec((C_in, C_prev), lambda b: (0, 0)),
            pl.BlockSpec((1, C_prev), lambda b: (0, 0)),
        ],
        out_specs=[
            pl.BlockSpec((1, Pp, Kp), lambda b: (b, 0, 0)),
            pl.BlockSpec((1, Np, C_prev), lambda b: (b, 0, 0)),
        ],
        out_shape=[
            jax.ShapeDtypeStruct((B, Pp, Kp), jnp.int32),
            jax.ShapeDtypeStruct((B, Np, C_prev), jnp.float32),
        ],
    )(repp, ptsp, fts, dense_W, dense_b.reshape(1, -1))
    return idx[:, :P, :K], fd


def _post_body(rep_ref, ptsr_ref, ftsr_ref,
               l1w_ref, l1b_ref, l2w_ref, l2b_ref,
               xw_ref, xb_ref, xd1w_ref, xd1b_ref, xd2w_ref, xd2b_ref,
               dwa_ref, dwb_ref, ba_ref, bb_ref, pwa_ref, pwb_ref,
               out_ref, *, K, P, C_mid, C_prev, dm):
    KK = K * K
    local = ptsr_ref[0] - rep_ref[0]
    local_flat = local.reshape(K * P, 3)
    l1 = _elu(jnp.dot(local_flat, l1w_ref[...],
                      preferred_element_type=jnp.float32) + l1b_ref[...])
    l2 = _elu(jnp.dot(l1, l2w_ref[...],
                      preferred_element_type=jnp.float32) + l2b_ref[...])
    a3 = l2.reshape(K, P, C_mid)
    b3 = ftsr_ref[0]

    X = jnp.zeros((P, KK), jnp.float32)
    for d in range(3):
        for k in range(K):
            X = X + local[k, :, d:d + 1] * xw_ref[d, k:k + 1, :]
    X = _elu(X + xb_ref[...])
    X = _elu(jnp.dot(X, xd1w_ref[...],
                     preferred_element_type=jnp.float32) + xd1b_ref[...])
    X = jnp.dot(X, xd2w_ref[...],
                preferred_element_type=jnp.float32) + xd2b_ref[...]

    fxa = []
    fxb = []
    for k in range(K):
        aa = jnp.zeros((P, C_mid), jnp.float32)
        bb_acc = jnp.zeros((P, C_prev), jnp.float32)
        for l in range(K):
            c = X[:, k * K + l:k * K + l + 1]
            aa = aa + c * a3[l]
            bb_acc = bb_acc + c * b3[l]
        fxa.append(aa)
        fxb.append(bb_acc)

    out = jnp.zeros((P, out_ref.shape[2]), jnp.float32)
    for m in range(dm):
        da = jnp.zeros((P, C_mid), jnp.float32)
        db = jnp.zeros((P, C_prev), jnp.float32)
        for k in range(K):
            da = da + fxa[k] * dwa_ref[m, k:k + 1, :]
            db = db + fxb[k] * dwb_ref[m, k:k + 1, :]
        da = da + ba_ref[m:m + 1, :]
        db = db + bb_ref[m:m + 1, :]
        out = out + jnp.dot(da, pwa_ref[m],
                            preferred_element_type=jnp.float32)
        out = out + jnp.dot(db, pwb_ref[m],
                            preferred_element_type=jnp.float32)
    out_ref[0] = _elu(out)


def _post(rep, pts_r, fts_r, p, li, K, P, C_out):
    g = lambda n: p['l%d_%s' % (li, n)]
    B = rep.shape[0]
    C_mid = g('lift1_W').shape[1]
    C_prev = fts_r.shape[3]
    dw_W = g('dw_W')
    Cc, dm, _ = dw_W.shape
    KK = K * K
    dwa = dw_W[:C_mid].transpose(1, 2, 0)
    dwb = dw_W[C_mid:].transpose(1, 2, 0)
    dwbias = g('dw_b').reshape(Cc, dm)
    ba = dwbias[:C_mid].T
    bb = dwbias[C_mid:].T
    pw = g('pw_W').reshape(Cc, dm, C_out)
    pwa = pw[:C_mid].transpose(1, 0, 2)
    pwb = pw[C_mid:].transpose(1, 0, 2)
    full = lambda arr: pl.BlockSpec(arr.shape, lambda b, n=arr.ndim: (0,) * n)
    weights = [g('lift1_W'), g('lift1_b').reshape(1, -1),
               g('lift2_W'), g('lift2_b').reshape(1, -1),
               g('xconv_W'), g('xconv_b').reshape(1, -1),
               g('xd1_W'), g('xd1_b').reshape(1, -1),
               g('xd2_W'), g('xd2_b').reshape(1, -1),
               dwa, dwb, ba, bb, pwa, pwb]
    out = pl.pallas_call(
        functools.partial(_post_body, K=K, P=P, C_mid=C_mid,
                          C_prev=C_prev, dm=dm),
        grid=(B,),
        in_specs=[
            pl.BlockSpec((1, P, 3), lambda b: (b, 0, 0)),
            pl.BlockSpec((1, K, P, 3), lambda b: (b, 0, 0, 0)),
            pl.BlockSpec((1, K, P, C_prev), lambda b: (b, 0, 0, 0)),
        ] + [full(w) for w in weights],
        out_specs=pl.BlockSpec((1, P, C_out), lambda b: (b, 0, 0)),
        out_shape=jax.ShapeDtypeStruct((B, P, C_out), jnp.float32),
    )(rep, pts_r, fts_r, *weights)
    return out


def _fc_head_kernel(fts_ref, w1_ref, b1_ref, w2_ref, b2_ref, w3_ref, b3_ref,
                    out_ref, *, B, P):
    f = fts_ref[...]
    f = jnp.maximum(jnp.dot(f, w1_ref[...],
                            preferred_element_type=jnp.float32) + b1_ref[...], 0.0)
    f = jnp.maximum(jnp.dot(f, w2_ref[...],
                            preferred_element_type=jnp.float32) + b2_ref[...], 0.0)
    logits = jnp.dot(f, w3_ref[...],
                     preferred_element_type=jnp.float32) + b3_ref[...]
    logits = jnp.mean(logits.reshape(B, P, -1), axis=1)
    m = jnp.max(logits, axis=-1, keepdims=True)
    s = jnp.log(jnp.sum(jnp.exp(logits - m), axis=-1, keepdims=True))
    out_ref[...] = logits - m - s


def _fc_head(fts, params):
    B, P, C = fts.shape
    return pl.pallas_call(
        functools.partial(_fc_head_kernel, B=B, P=P),
        out_shape=jax.ShapeDtypeStruct((B, 10), jnp.float32),
    )(fts.reshape(B * P, C),
      params['fc1_W'], params['fc1_b'].reshape(1, -1),
      params['fc2_W'], params['fc2_b'].reshape(1, -1),
      params['fc3_W'], params['fc3_b'].reshape(1, -1))


def _gather(x, idx):
    return jax.vmap(lambda a, i: a[i])(x, idx)


def _layer(li, params, pts, fts, C_in, C_out, K, D, P):
    g = lambda n: params['l%d_%s' % (li, n)]
    B, Np, _ = pts.shape
    if 0 < P < Np:
        sel = _SEL_CONST[(li, Np)][:P]
        rep = pts[:, sel, :]
    else:
        rep = pts
        P = Np
    idx, fts_d = _knn_dense(rep, pts, fts, g('dense_W'), g('dense_b'), K, D)
    pts_r = jnp.swapaxes(_gather(pts, idx), 1, 2)
    fts_r = jnp.swapaxes(_gather(fts_d, idx), 1, 2)
    out = _post(rep, pts_r, fts_r, params, li, K, P, C_out)
    return rep, out


def kernel(pts, fts, params):
    cur_pts, cur_fts = pts, fts
    for li, (C_in, C_out, K, D, P) in enumerate(_LAYER_CFG):
        cur_pts, cur_fts = _layer(li, params, cur_pts, cur_fts,
                                  C_in, C_out, K, D, P)
    return _fc_head(cur_fts, params)


# speedup vs baseline: 2.6921x; 2.6921x over previous
import functools

import jax
import jax.numpy as jnp
import numpy as np
from jax import lax
from jax.experimental import pallas as pl
from jax.experimental.pallas import tpu as pltpu
from jax.experimental.pallas import tpu_sc as plsc

_LAYER_CFG = [
    (1, 32, 8, 1, 256),
    (32, 64, 8, 2, 256),
    (64, 96, 8, 4, 256),
    (96, 128, 12, 4, 120),
    (128, 160, 12, 6, 120),
]

_SEL0 = (59,643,95,698,716,968,175,251,890,383,935,269,221,694,77,111,229,641,391,875,35,660,231,332,381,410,299,745,463,904,987,446,702,202,135,174,441,805,270,102,957,813,970,669,126,928,795,304,448,940,609,727,119,528,309,938,512,429,110,590,912,454,179,378,107,1013,551,533,359,676,431,1018,874,843,877,66,97,453,812,458,550,497,394,691,406,866,983,695,443,294,978,209,700,470,699,816,291,505,607,177,476,422,57,868,542,338,836,280,493,589,134,925,548,1006,638,242,827,593,6,639,986,581,444,411,211,734,559,389,388,393,823,455,591,30,164,626,257,278,984,139,307,704,588,28,757,824,683,33,63,781,769,785,608,374,173,764,418,697,884,213,366,815,449,599,861,1,335,247,482,644,445,308,993,521,828,680,571,404,461,932,180,316,58,793,623,540,2,1001,790,943,491,566,53,634,679,184,37,678,556,666,70,261,370,435,290,846,222,597,492,901,888,825,45,125,17,895,754,424,436,746,547,365,1022,96,911,166,156,32,741,879,48,725,954,671,228,254,165,153,301,457,423,944,118,905,194,931,56,76,14,918,594,713,69,155,648,948)

_SEL3 = (59,28,216,51,0,47,76,197,140,128,27,176,48,45,50,180,29,15,53,243,141,126,113,1,95,251,78,238,212,190,25,105,131,196,71,159,73,206,117,154,97,120,82,227,249,63,157,252,75,164,185,240,85,16,13,152,112,42,188,115,133,98,239,61,225,231,46,103,19,230,88,170,93,32,122,255,191,11,235,153,179,169,139,160,67,134,211,81,18,254,68,220,56,245,177,70,165,8,38,23,3,79,234,44,138,26,142,39,213,222,43,22,144,182,108,201,60,219,119,187)

_SEL_CONST = {
    (0, 1024): np.asarray(_SEL0, dtype=np.int32),
    (3, 256): np.asarray(_SEL3, dtype=np.int32),
}

_NUM_WORKERS = 32


def _elu(x):
    return jnp.where(x > 0, x, jnp.exp(x) - 1.0)


def _pad_to(x, axis, size, value):
    pad = size - x.shape[axis]
    if pad <= 0:
        return x
    widths = [(0, 0)] * x.ndim
    widths[axis] = (0, pad)
    return jnp.pad(x, widths, constant_values=value)


def _knn_body(rep_ref, pts_ref, pts2_ref, fts_ref, dw_ref, db_ref,
              idx_ref, tbl_ref, *, K, D, C_prev, W):
    Pp = rep_ref.shape[1]
    Npp = pts_ref.shape[2]
    Np = pts2_ref.shape[1]
    rx = rep_ref[0, :, 0:1]
    ry = rep_ref[0, :, 1:2]
    rz = rep_ref[0, :, 2:3]
    px = pts_ref[0, 0:1, :]
    py = pts_ref[0, 1:2, :]
    pz = pts_ref[0, 2:3, :]
    dx = rx - px
    dy = ry - py
    dz = rz - pz
    d2 = dx * dx + dy * dy + dz * dz
    iota_n = jax.lax.broadcasted_iota(jnp.int32, (Pp, Npp), 1)
    lane_k = jax.lax.broadcasted_iota(jnp.int32, (Pp, idx_ref.shape[2]), 1)
    acc = jnp.zeros((Pp, idx_ref.shape[2]), jnp.int32)
    inf = jnp.float32(jnp.inf)
    for k in range(D * K + 1):
        m = jnp.min(d2, axis=1, keepdims=True)
        cand = jnp.where(d2 == m, iota_n, jnp.int32(Npp))
        am = jnp.min(cand, axis=1, keepdims=True)
        if k >= 1 and (k - 1) % D == 0:
            acc = jnp.where(lane_k == (k - 1) // D, am, acc)
        if k < D * K:
            d2 = jnp.where(iota_n == am, inf, d2)
    idx_ref[0] = acc
    fd = _elu(jnp.dot(fts_ref[0], dw_ref[...],
                      preferred_element_type=jnp.float32) + db_ref[...])
    tbl_ref[0] = jnp.zeros((Np, W), jnp.float32)
    tbl_ref[0, :, 0:C_prev] = fd
    tbl_ref[0, :, C_prev:C_prev + 3] = pts2_ref[0]


def _knn_dense(rep, pts, fts, dense_W, dense_b, K, D, W):
    B, P, _ = rep.shape
    Np = pts.shape[1]
    C_in = fts.shape[2]
    C_prev = dense_W.shape[1]
    Pp = max(128, ((P + 7) // 8) * 8)
    Npp = max(128, ((Np + 127) // 128) * 128)
    Kp = 16
    repp = _pad_to(rep, 1, Pp, 0.0)
    ptsp = _pad_to(jnp.swapaxes(pts, 1, 2), 2, Npp, 1e30)
    idx, tbl = pl.pallas_call(
        functools.partial(_knn_body, K=K, D=D, C_prev=C_prev, W=W),
        grid=(B,),
        in_specs=[
            pl.BlockSpec((1, Pp, 3), lambda b: (b, 0, 0)),
            pl.BlockSpec((1, 3, Npp), lambda b: (b, 0, 0)),
            pl.BlockSpec((1, Np, 3), lambda b: (b, 0, 0)),
            pl.BlockSpec((1, Np, C_in), lambda b: (b, 0, 0)),
            pl.BlockSpec((C_in, C_prev), lambda b: (0, 0)),
            pl.BlockSpec((1, C_prev), lambda b: (0, 0)),
        ],
        out_specs=[
            pl.BlockSpec((1, Pp, Kp), lambda b: (b, 0, 0)),
            pl.BlockSpec((1, Np, W), lambda b: (b, 0, 0)),
        ],
        out_shape=[
            jax.ShapeDtypeStruct((B, Pp, Kp), jnp.int32),
            jax.ShapeDtypeStruct((B, Np, W), jnp.float32),
        ],
    )(repp, ptsp, pts, fts, dense_W, dense_b.reshape(1, -1))
    return idx[:, :P, :K], tbl


def _sc_gather(table, idx_flat, W, nchunks, chunk, nsub):
    N = idx_flat.shape[0]
    assert N == _NUM_WORKERS * nchunks * chunk and chunk == nsub * 128
    assert W == 128
    per_tile = nchunks * chunk
    idx2d = idx_flat.reshape(N // 128, 128)
    mesh = plsc.VectorSubcoreMesh(core_axis_name="c", subcore_axis_name="s")

    @functools.partial(
        pl.kernel, mesh=mesh,
        out_type=jax.ShapeDtypeStruct((N, W), jnp.float32),
        scratch_types=[
            pltpu.VMEM((per_tile // 128, 128), jnp.int32),
            pltpu.VMEM((chunk, W), jnp.float32),
            pltpu.SemaphoreType.DMA,
        ],
    )
    def gather_k(table_hbm, idx_hbm, out_hbm, idx_v, rows_v, sem):
        wid = lax.axis_index("s") * 2 + lax.axis_index("c")
        base = wid * per_tile
        pltpu.sync_copy(
            idx_hbm.at[pl.ds(pl.multiple_of(base // 128, 8), per_tile // 128)],
            idx_v)
        for c in range(nchunks):
            copies = []
            for j in range(nsub):
                copies.append(pltpu.async_copy(
                    table_hbm.at[idx_v.at[c * nsub + j]],
                    rows_v.at[pl.ds(j * 128, 128)], sem))
            for cp in copies:
                cp.wait()
            pltpu.sync_copy(
                rows_v,
                out_hbm.at[pl.ds(pl.multiple_of(base + c * chunk, 512), chunk)])

    return gather_k(table, idx2d)


_SC_CFG = [
    (128, 4, 512, 4),
    (128, 4, 512, 4),
    (128, 4, 512, 4),
    (128, 4, 512, 4),
    (128, 4, 512, 4),
]


def _post_body(rep_ref, gath_ref,
               l1w_ref, l1b_ref, l2w_ref, l2b_ref,
               xw_ref, xb_ref, xd1w_ref, xd1b_ref, xd2w_ref, xd2b_ref,
               dwa_ref, dwb_ref, ba_ref, bb_ref, pwa_ref, pwb_ref,
               out_ref, *, K, P, C_mid, C_prev, dm):
    KK = K * K
    gath = gath_ref[0]
    b3 = gath[:, :, 0:C_prev]
    local = gath[:, :, C_prev:C_prev + 3] - rep_ref[0]
    local_flat = local.reshape(K * P, 3)
    l1 = _elu(jnp.dot(local_flat, l1w_ref[...],
                      preferred_element_type=jnp.float32) + l1b_ref[...])
    l2 = _elu(jnp.dot(l1, l2w_ref[...],
                      preferred_element_type=jnp.float32) + l2b_ref[...])
    a3 = l2.reshape(K, P, C_mid)

    X = jnp.zeros((P, KK), jnp.float32)
    for d in range(3):
        for k in range(K):
            X = X + local[k, :, d:d + 1] * xw_ref[d, k:k + 1, :]
    X = _elu(X + xb_ref[...])
    X = _elu(jnp.dot(X, xd1w_ref[...],
                     preferred_element_type=jnp.float32) + xd1b_ref[...])
    X = jnp.dot(X, xd2w_ref[...],
                preferred_element_type=jnp.float32) + xd2b_ref[...]

    fxa = []
    fxb = []
    for k in range(K):
        aa = jnp.zeros((P, C_mid), jnp.float32)
        bb_acc = jnp.zeros((P, C_prev), jnp.float32)
        for l in range(K):
            c = X[:, k * K + l:k * K + l + 1]
            aa = aa + c * a3[l]
            bb_acc = bb_acc + c * b3[l]
        fxa.append(aa)
        fxb.append(bb_acc)

    out = jnp.zeros((P, out_ref.shape[2]), jnp.float32)
    for m in range(dm):
        da = jnp.zeros((P, C_mid), jnp.float32)
        db = jnp.zeros((P, C_prev), jnp.float32)
        for k in range(K):
            da = da + fxa[k] * dwa_ref[m, k:k + 1, :]
            db = db + fxb[k] * dwb_ref[m, k:k + 1, :]
        da = da + ba_ref[m:m + 1, :]
        db = db + bb_ref[m:m + 1, :]
        out = out + jnp.dot(da, pwa_ref[m],
                            preferred_element_type=jnp.float32)
        out = out + jnp.dot(db, pwb_ref[m],
                            preferred_element_type=jnp.float32)
    out_ref[0] = _elu(out)


def _post(rep, gath, p, li, K, P, C_out, C_prev):
    g = lambda n: p['l%d_%s' % (li, n)]
    B = rep.shape[0]
    W = gath.shape[3]
    C_mid = g('lift1_W').shape[1]
    dw_W = g('dw_W')
    Cc, dm, _ = dw_W.shape
    dwa = dw_W[:C_mid].transpose(1, 2, 0)
    dwb = dw_W[C_mid:].transpose(1, 2, 0)
    dwbias = g('dw_b').reshape(Cc, dm)
    ba = dwbias[:C_mid].T
    bb = dwbias[C_mid:].T
    pw = g('pw_W').reshape(Cc, dm, C_out)
    pwa = pw[:C_mid].transpose(1, 0, 2)
    pwb = pw[C_mid:].transpose(1, 0, 2)
    full = lambda arr: pl.BlockSpec(arr.shape, lambda b, n=arr.ndim: (0,) * n)
    weights = [g('lift1_W'), g('lift1_b').reshape(1, -1),
               g('lift2_W'), g('lift2_b').reshape(1, -1),
               g('xconv_W'), g('xconv_b').reshape(1, -1),
               g('xd1_W'), g('xd1_b').reshape(1, -1),
               g('xd2_W'), g('xd2_b').reshape(1, -1),
               dwa, dwb, ba, bb, pwa, pwb]
    out = pl.pallas_call(
        functools.partial(_post_body, K=K, P=P, C_mid=C_mid,
                          C_prev=C_prev, dm=dm),
        grid=(B,),
        in_specs=[
            pl.BlockSpec((1, P, 3), lambda b: (b, 0, 0)),
            pl.BlockSpec((1, K, P, W), lambda b: (b, 0, 0, 0)),
        ] + [full(w) for w in weights],
        out_specs=pl.BlockSpec((1, P, C_out), lambda b: (b, 0, 0)),
        out_shape=jax.ShapeDtypeStruct((B, P, C_out), jnp.float32),
    )(rep, gath, *weights)
    return out


def _fc_head_kernel(fts_ref, w1_ref, b1_ref, w2_ref, b2_ref, w3_ref, b3_ref,
                    out_ref, *, B, P):
    f = fts_ref[...]
    f = jnp.maximum(jnp.dot(f, w1_ref[...],
                            preferred_element_type=jnp.float32) + b1_ref[...], 0.0)
    f = jnp.maximum(jnp.dot(f, w2_ref[...],
                            preferred_element_type=jnp.float32) + b2_ref[...], 0.0)
    logits = jnp.dot(f, w3_ref[...],
                     preferred_element_type=jnp.float32) + b3_ref[...]
    logits = jnp.mean(logits.reshape(B, P, -1), axis=1)
    m = jnp.max(logits, axis=-1, keepdims=True)
    s = jnp.log(jnp.sum(jnp.exp(logits - m), axis=-1, keepdims=True))
    out_ref[...] = logits - m - s


def _fc_head(fts, params):
    B, P, C = fts.shape
    return pl.pallas_call(
        functools.partial(_fc_head_kernel, B=B, P=P),
        out_shape=jax.ShapeDtypeStruct((B, 10), jnp.float32),
    )(fts.reshape(B * P, C),
      params['fc1_W'], params['fc1_b'].reshape(1, -1),
      params['fc2_W'], params['fc2_b'].reshape(1, -1),
      params['fc3_W'], params['fc3_b'].reshape(1, -1))


def _layer(li, params, pts, fts, C_in, C_out, K, D, P):
    g = lambda n: params['l%d_%s' % (li, n)]
    B, Np, _ = pts.shape
    if 0 < P < Np:
        sel = _SEL_CONST[(li, Np)][:P]
        rep = pts[:, sel, :]
    else:
        rep = pts
        P = Np
    C_prev = g('dense_W').shape[1]
    W, nchunks, chunk, nsub = _SC_CFG[li]
    idx, tbl = _knn_dense(rep, pts, fts, g('dense_W'), g('dense_b'), K, D, W)
    offs = (jnp.arange(B, dtype=jnp.int32) * Np)[:, None, None]
    flat_idx = (jnp.swapaxes(idx, 1, 2) + offs).reshape(-1)
    N = B * K * P
    Ntot = _NUM_WORKERS * nchunks * chunk
    flat_idx = _pad_to(flat_idx, 0, Ntot, 0)
    rows = _sc_gather(tbl.reshape(B * Np, W), flat_idx, W, nchunks, chunk, nsub)
    gath = rows[:N].reshape(B, K, P, W)
    out = _post(rep, gath, params, li, K, P, C_out, C_prev)
    return rep, out


def kernel(pts, fts, params):
    cur_pts, cur_fts = pts, fts
    for li, (C_in, C_out, K, D, P) in enumerate(_LAYER_CFG):
        cur_pts, cur_fts = _layer(li, params, cur_pts, cur_fts,
                                  C_in, C_out, K, D, P)
    return _fc_head(cur_fts, params)


# speedup vs baseline: 4.4892x; 1.6675x over previous
import functools

import jax
import jax.numpy as jnp
import numpy as np
from jax import lax
from jax.experimental import pallas as pl
from jax.experimental.pallas import tpu as pltpu
from jax.experimental.pallas import tpu_sc as plsc

_LAYER_CFG = [
    (1, 32, 8, 1, 256),
    (32, 64, 8, 2, 256),
    (64, 96, 8, 4, 256),
    (96, 128, 12, 4, 120),
    (128, 160, 12, 6, 120),
]

_SEL0 = (59,643,95,698,716,968,175,251,890,383,935,269,221,694,77,111,229,641,391,875,35,660,231,332,381,410,299,745,463,904,987,446,702,202,135,174,441,805,270,102,957,813,970,669,126,928,795,304,448,940,609,727,119,528,309,938,512,429,110,590,912,454,179,378,107,1013,551,533,359,676,431,1018,874,843,877,66,97,453,812,458,550,497,394,691,406,866,983,695,443,294,978,209,700,470,699,816,291,505,607,177,476,422,57,868,542,338,836,280,493,589,134,925,548,1006,638,242,827,593,6,639,986,581,444,411,211,734,559,389,388,393,823,455,591,30,164,626,257,278,984,139,307,704,588,28,757,824,683,33,63,781,769,785,608,374,173,764,418,697,884,213,366,815,449,599,861,1,335,247,482,644,445,308,993,521,828,680,571,404,461,932,180,316,58,793,623,540,2,1001,790,943,491,566,53,634,679,184,37,678,556,666,70,261,370,435,290,846,222,597,492,901,888,825,45,125,17,895,754,424,436,746,547,365,1022,96,911,166,156,32,741,879,48,725,954,671,228,254,165,153,301,457,423,944,118,905,194,931,56,76,14,918,594,713,69,155,648,948)

_SEL3 = (59,28,216,51,0,47,76,197,140,128,27,176,48,45,50,180,29,15,53,243,141,126,113,1,95,251,78,238,212,190,25,105,131,196,71,159,73,206,117,154,97,120,82,227,249,63,157,252,75,164,185,240,85,16,13,152,112,42,188,115,133,98,239,61,225,231,46,103,19,230,88,170,93,32,122,255,191,11,235,153,179,169,139,160,67,134,211,81,18,254,68,220,56,245,177,70,165,8,38,23,3,79,234,44,138,26,142,39,213,222,43,22,144,182,108,201,60,219,119,187)

_SEL_CONST = {
    (0, 1024): np.asarray(_SEL0, dtype=np.int32),
    (3, 256): np.asarray(_SEL3, dtype=np.int32),
}

_NUM_WORKERS = 32


def _elu(x):
    return jnp.where(x > 0, x, jnp.exp(x) - 1.0)


def _pad_to(x, axis, size, value):
    pad = size - x.shape[axis]
    if pad <= 0:
        return x
    widths = [(0, 0)] * x.ndim
    widths[axis] = (0, pad)
    return jnp.pad(x, widths, constant_values=value)


def _knn_body(rep_ref, pts_ref, pts2_ref, fts_ref, dw_ref, db_ref,
              idx_ref, tbl_ref, *, K, D, C_prev, W):
    Pp = rep_ref.shape[1]
    Npp = pts_ref.shape[2]
    Np = pts2_ref.shape[1]
    rx = rep_ref[0, :, 0:1]
    ry = rep_ref[0, :, 1:2]
    rz = rep_ref[0, :, 2:3]
    px = pts_ref[0, 0:1, :]
    py = pts_ref[0, 1:2, :]
    pz = pts_ref[0, 2:3, :]
    dx = rx - px
    dy = ry - py
    dz = rz - pz
    d2 = dx * dx + dy * dy + dz * dz
    iota_n = jax.lax.broadcasted_iota(jnp.int32, (Pp, Npp), 1)
    lane_k = jax.lax.broadcasted_iota(jnp.int32, (Pp, idx_ref.shape[2]), 1)
    acc = jnp.zeros((Pp, idx_ref.shape[2]), jnp.int32)
    inf = jnp.float32(jnp.inf)
    for k in range(D * K + 1):
        m = jnp.min(d2, axis=1, keepdims=True)
        cand = jnp.where(d2 == m, iota_n, jnp.int32(Npp))
        am = jnp.min(cand, axis=1, keepdims=True)
        if k >= 1 and (k - 1) % D == 0:
            acc = jnp.where(lane_k == (k - 1) // D, am, acc)
        if k < D * K:
            d2 = jnp.where(iota_n == am, inf, d2)
    idx_ref[0] = acc
    fd = _elu(jnp.dot(fts_ref[0], dw_ref[...],
                      preferred_element_type=jnp.float32) + db_ref[...])
    tbl_ref[0] = jnp.zeros((Np, W), jnp.float32)
    tbl_ref[0, :, 0:C_prev] = fd
    tbl_ref[0, :, C_prev:C_prev + 3] = pts2_ref[0]


def _knn_dense(rep, pts, fts, dense_W, dense_b, K, D, W):
    B, P, _ = rep.shape
    Np = pts.shape[1]
    C_in = fts.shape[2]
    C_prev = dense_W.shape[1]
    Pp = max(128, ((P + 7) // 8) * 8)
    Npp = max(128, ((Np + 127) // 128) * 128)
    Kp = 16
    repp = _pad_to(rep, 1, Pp, 0.0)
    ptsp = _pad_to(jnp.swapaxes(pts, 1, 2), 2, Npp, 1e30)
    idx, tbl = pl.pallas_call(
        functools.partial(_knn_body, K=K, D=D, C_prev=C_prev, W=W),
        grid=(B,),
        in_specs=[
            pl.BlockSpec((1, Pp, 3), lambda b: (b, 0, 0)),
            pl.BlockSpec((1, 3, Npp), lambda b: (b, 0, 0)),
            pl.BlockSpec((1, Np, 3), lambda b: (b, 0, 0)),
            pl.BlockSpec((1, Np, C_in), lambda b: (b, 0, 0)),
            pl.BlockSpec((C_in, C_prev), lambda b: (0, 0)),
            pl.BlockSpec((1, C_prev), lambda b: (0, 0)),
        ],
        out_specs=[
            pl.BlockSpec((1, Pp, Kp), lambda b: (b, 0, 0)),
            pl.BlockSpec((1, Np, W), lambda b: (b, 0, 0)),
        ],
        out_shape=[
            jax.ShapeDtypeStruct((B, Pp, Kp), jnp.int32),
            jax.ShapeDtypeStruct((B, Np, W), jnp.float32),
        ],
    )(repp, ptsp, pts, fts, dense_W, dense_b.reshape(1, -1))
    return idx[:, :P, :K], tbl


def _sc_gather(table, idx_flat, W, nchunks, chunk, nsub):
    N = idx_flat.shape[0]
    assert N == _NUM_WORKERS * nchunks * chunk and chunk == nsub * 128
    assert W == 128
    per_tile = nchunks * chunk
    nrows = per_tile // 128
    idx3d = idx_flat.reshape(_NUM_WORKERS, nrows, 128)
    mesh = plsc.VectorSubcoreMesh(core_axis_name="c", subcore_axis_name="s")

    @functools.partial(
        pl.kernel, mesh=mesh,
        out_type=jax.ShapeDtypeStruct((N, W), jnp.float32),
        scratch_types=[
            pltpu.VMEM((nrows, 128), jnp.int32),
            pltpu.VMEM((chunk, W), jnp.float32),
            pltpu.SemaphoreType.DMA,
        ],
    )
    def gather_k(table_hbm, idx_hbm, out_hbm, idx_v, rows_v, sem):
        wid = lax.axis_index("s") * 2 + lax.axis_index("c")
        base = wid * per_tile
        pltpu.sync_copy(idx_hbm.at[wid], idx_v)
        for c in range(nchunks):
            copies = []
            for j in range(nsub):
                copies.append(pltpu.async_copy(
                    table_hbm.at[idx_v.at[c * nsub + j]],
                    rows_v.at[pl.ds(j * 128, 128)], sem))
            for cp in copies:
                cp.wait()
            pltpu.sync_copy(
                rows_v,
                out_hbm.at[pl.ds(pl.multiple_of(base + c * chunk, 512), chunk)])

    return gather_k(table, idx3d)


_SC_CFG = [
    (128, 4, 512, 4),
    (128, 4, 512, 4),
    (128, 4, 512, 4),
    (128, 3, 512, 4),
    (128, 3, 512, 4),
]


def _post_body(rep_ref, gath_ref,
               l1w_ref, l1b_ref, l2w_ref, l2b_ref,
               xw_ref, xb_ref, xd1w_ref, xd1b_ref, xd2w_ref, xd2b_ref,
               dwa_ref, dwb_ref, ba_ref, bb_ref, pwa_ref, pwb_ref,
               out_ref, *, K, P, C_mid, C_prev, dm):
    KK = K * K
    gath = gath_ref[0]
    b3 = gath[:, :, 0:C_prev]
    local = gath[:, :, C_prev:C_prev + 3] - rep_ref[0]
    local_flat = local.reshape(K * P, 3)
    l1 = _elu(jnp.dot(local_flat, l1w_ref[...],
                      preferred_element_type=jnp.float32) + l1b_ref[...])
    l2 = _elu(jnp.dot(l1, l2w_ref[...],
                      preferred_element_type=jnp.float32) + l2b_ref[...])
    a3 = l2.reshape(K, P, C_mid)

    X = jnp.zeros((P, KK), jnp.float32)
    for d in range(3):
        for k in range(K):
            X = X + local[k, :, d:d + 1] * xw_ref[d, k:k + 1, :]
    X = _elu(X + xb_ref[...])
    X = _elu(jnp.dot(X, xd1w_ref[...],
                     preferred_element_type=jnp.float32) + xd1b_ref[...])
    X = jnp.dot(X, xd2w_ref[...],
                preferred_element_type=jnp.float32) + xd2b_ref[...]

    fxa = []
    fxb = []
    for k in range(K):
        aa = jnp.zeros((P, C_mid), jnp.float32)
        bb_acc = jnp.zeros((P, C_prev), jnp.float32)
        for l in range(K):
            c = X[:, k * K + l:k * K + l + 1]
            aa = aa + c * a3[l]
            bb_acc = bb_acc + c * b3[l]
        fxa.append(aa)
        fxb.append(bb_acc)

    out = jnp.zeros((P, out_ref.shape[2]), jnp.float32)
    for m in range(dm):
        da = jnp.zeros((P, C_mid), jnp.float32)
        db = jnp.zeros((P, C_prev), jnp.float32)
        for k in range(K):
            da = da + fxa[k] * dwa_ref[m, k:k + 1, :]
            db = db + fxb[k] * dwb_ref[m, k:k + 1, :]
        da = da + ba_ref[m:m + 1, :]
        db = db + bb_ref[m:m + 1, :]
        out = out + jnp.dot(da, pwa_ref[m],
                            preferred_element_type=jnp.float32)
        out = out + jnp.dot(db, pwb_ref[m],
                            preferred_element_type=jnp.float32)
    out_ref[0] = _elu(out)


def _post(rep, gath, p, li, K, P, C_out, C_prev):
    g = lambda n: p['l%d_%s' % (li, n)]
    B = rep.shape[0]
    W = gath.shape[3]
    C_mid = g('lift1_W').shape[1]
    dw_W = g('dw_W')
    Cc, dm, _ = dw_W.shape
    dwa = dw_W[:C_mid].transpose(1, 2, 0)
    dwb = dw_W[C_mid:].transpose(1, 2, 0)
    dwbias = g('dw_b').reshape(Cc, dm)
    ba = dwbias[:C_mid].T
    bb = dwbias[C_mid:].T
    pw = g('pw_W').reshape(Cc, dm, C_out)
    pwa = pw[:C_mid].transpose(1, 0, 2)
    pwb = pw[C_mid:].transpose(1, 0, 2)
    full = lambda arr: pl.BlockSpec(arr.shape, lambda b, n=arr.ndim: (0,) * n)
    weights = [g('lift1_W'), g('lift1_b').reshape(1, -1),
               g('lift2_W'), g('lift2_b').reshape(1, -1),
               g('xconv_W'), g('xconv_b').reshape(1, -1),
               g('xd1_W'), g('xd1_b').reshape(1, -1),
               g('xd2_W'), g('xd2_b').reshape(1, -1),
               dwa, dwb, ba, bb, pwa, pwb]
    out = pl.pallas_call(
        functools.partial(_post_body, K=K, P=P, C_mid=C_mid,
                          C_prev=C_prev, dm=dm),
        grid=(B,),
        in_specs=[
            pl.BlockSpec((1, P, 3), lambda b: (b, 0, 0)),
            pl.BlockSpec((1, K, P, W), lambda b: (b, 0, 0, 0)),
        ] + [full(w) for w in weights],
        out_specs=pl.BlockSpec((1, P, C_out), lambda b: (b, 0, 0)),
        out_shape=jax.ShapeDtypeStruct((B, P, C_out), jnp.float32),
    )(rep, gath, *weights)
    return out


def _fc_head_kernel(fts_ref, w1_ref, b1_ref, w2_ref, b2_ref, w3_ref, b3_ref,
                    out_ref, *, B, P):
    f = fts_ref[...]
    f = jnp.maximum(jnp.dot(f, w1_ref[...],
                            preferred_element_type=jnp.float32) + b1_ref[...], 0.0)
    f = jnp.maximum(jnp.dot(f, w2_ref[...],
                            preferred_element_type=jnp.float32) + b2_ref[...], 0.0)
    logits = jnp.dot(f, w3_ref[...],
                     preferred_element_type=jnp.float32) + b3_ref[...]
    logits = jnp.mean(logits.reshape(B, P, -1), axis=1)
    m = jnp.max(logits, axis=-1, keepdims=True)
    s = jnp.log(jnp.sum(jnp.exp(logits - m), axis=-1, keepdims=True))
    out_ref[...] = logits - m - s


def _fc_head(fts, params):
    B, P, C = fts.shape
    return pl.pallas_call(
        functools.partial(_fc_head_kernel, B=B, P=P),
        out_shape=jax.ShapeDtypeStruct((B, 10), jnp.float32),
    )(fts.reshape(B * P, C),
      params['fc1_W'], params['fc1_b'].reshape(1, -1),
      params['fc2_W'], params['fc2_b'].reshape(1, -1),
      params['fc3_W'], params['fc3_b'].reshape(1, -1))


def _layer(li, params, pts, fts, C_in, C_out, K, D, P):
    g = lambda n: params['l%d_%s' % (li, n)]
    B, Np, _ = pts.shape
    if 0 < P < Np:
        sel = _SEL_CONST[(li, Np)][:P]
        rep = pts[:, sel, :]
    else:
        rep = pts
        P = Np
    C_prev = g('dense_W').shape[1]
    W, nchunks, chunk, nsub = _SC_CFG[li]
    idx, tbl = _knn_dense(rep, pts, fts, g('dense_W'), g('dense_b'), K, D, W)
    offs = (jnp.arange(B, dtype=jnp.int32) * Np)[:, None, None]
    flat_idx = (jnp.swapaxes(idx, 1, 2) + offs).reshape(-1)
    N = B * K * P
    Ntot = _NUM_WORKERS * nchunks * chunk
    if Ntot > N:
        pad_idx = jnp.arange(Ntot - N, dtype=jnp.int32) % (B * Np)
        flat_idx = jnp.concatenate([flat_idx, pad_idx])
    rows = _sc_gather(tbl.reshape(B * Np, W), flat_idx, W, nchunks, chunk, nsub)
    gath = rows[:N].reshape(B, K, P, W)
    out = _post(rep, gath, params, li, K, P, C_out, C_prev)
    return rep, out


def kernel(pts, fts, params):
    cur_pts, cur_fts = pts, fts
    for li, (C_in, C_out, K, D, P) in enumerate(_LAYER_CFG):
        cur_pts, cur_fts = _layer(li, params, cur_pts, cur_fts,
                                  C_in, C_out, K, D, P)
    return _fc_head(cur_fts, params)
